# Initial kernel scaffold; baseline (speedup 1.0000x reference)
import jax
import jax.numpy as jnp
from jax.experimental import pallas as pl


def kernel(x, edge_index, edge_values, positions, Wqkv, bqkv, Wproj, bproj, Wgate, bgate):
    raise NotImplementedError("write your pallas kernel here")


# speedup vs baseline: 19.4851x; 19.4851x over previous
import functools
import math

import jax
import jax.numpy as jnp
from jax import lax
from jax.experimental import pallas as pl
from jax.experimental.pallas import tpu as pltpu
from jax.experimental.pallas import tpu_sc as plsc

DIM = 128
BS = 32
NH = 4
HD = DIM // NH

_NW = 32
_CAP = 1024
_CAPP = _CAP + 16
_SENT = 2**31 - 1
_BLK_PER_W = 10
_ROWS_W = _BLK_PER_W * BS
_FLATW = _ROWS_W * DIM
_NPAD = _NW * _ROWS_W


def _compact_edges(rows, cols, vals):
    e = rows.shape[0]
    chunk = e // _NW
    nvec = chunk // 16
    mesh = plsc.VectorSubcoreMesh(core_axis_name="c", subcore_axis_name="s")

    @functools.partial(
        pl.kernel,
        out_type=(
            jax.ShapeDtypeStruct((_NW, _CAPP), jnp.int32),
            jax.ShapeDtypeStruct((_NW, _CAPP), jnp.float32),
        ),
        mesh=mesh,
        scratch_types=[
            pltpu.VMEM((chunk,), jnp.int32),
            pltpu.VMEM((chunk,), jnp.int32),
            pltpu.VMEM((chunk,), jnp.float32),
            pltpu.VMEM((_CAPP,), jnp.int32),
            pltpu.VMEM((_CAPP,), jnp.float32),
        ],
        compiler_params=pltpu.CompilerParams(needs_layout_passes=False),
    )
    def k(rows_hbm, cols_hbm, vals_hbm, idx_hbm, val_hbm, rv, cv, vv, ib, vb):
        wid = lax.axis_index("s") * 2 + lax.axis_index("c")
        base = wid * chunk
        pltpu.sync_copy(rows_hbm.at[pl.ds(base, chunk)], rv)
        pltpu.sync_copy(cols_hbm.at[pl.ds(base, chunk)], cv)
        pltpu.sync_copy(vals_hbm.at[pl.ds(base, chunk)], vv)

        sent = jnp.full((16,), _SENT, jnp.int32)

        def init(i, carry):
            ib[pl.ds(i * 16, 16)] = sent
            return carry

        lax.fori_loop(0, _CAPP // 16, init, jnp.int32(0))

        def body(i, off):
            r = rv[pl.ds(i * 16, 16)]
            c = cv[pl.ds(i * 16, 16)]
            v = vv[pl.ds(i * 16, 16)]
            br = r >> 5
            intra = br == (c >> 5)
            idx = r * DIM + (br & 3) * BS + (c & 31)
            ii = intra.astype(jnp.int32)
            offv = jnp.full((16,), off, jnp.int32)
            trash = jnp.full((16,), _CAPP - 1, jnp.int32)
            pos = offv + plsc.cumsum(ii) - ii
            tgt = jnp.where(intra, pos, trash)
            plsc.store_scatter(ib, [tgt], jnp.where(intra, idx, sent))
            plsc.store_scatter(vb, [tgt], v)
            cnt = jnp.sum(ii)
            return jnp.minimum(off + cnt, jnp.int32(_CAP))

        lax.fori_loop(0, nvec, body, jnp.int32(0))
        pltpu.sync_copy(ib, idx_hbm.at[wid])
        pltpu.sync_copy(vb, val_hbm.at[wid])

    return k(rows, cols, vals)


def _scatter_bias(idx_arr, val_arr):
    mesh = plsc.VectorSubcoreMesh(core_axis_name="c", subcore_axis_name="s")

    @functools.partial(
        pl.kernel,
        out_type=jax.ShapeDtypeStruct((_NPAD * DIM,), jnp.float32),
        mesh=mesh,
        scratch_types=[
            pltpu.VMEM((_CAPP,), jnp.int32),
            pltpu.VMEM((_CAPP,), jnp.float32),
            pltpu.VMEM((_FLATW + 16,), jnp.float32),
        ],
        compiler_params=pltpu.CompilerParams(needs_layout_passes=False),
    )
    def k(idx_hbm, val_hbm, wd_hbm, iv, vv, wb):
        wid = lax.axis_index("s") * 2 + lax.axis_index("c")
        lo = wid * _FLATW
        nanv = jnp.full((16,), jnp.nan, jnp.float32)

        def init(i, carry):
            wb[pl.ds(i * 16, 16)] = nanv
            return carry

        lax.fori_loop(0, (_FLATW + 16) // 16, init, jnp.int32(0))

        def outer(src, carry):
            pltpu.sync_copy(idx_hbm.at[src], iv)
            pltpu.sync_copy(val_hbm.at[src], vv)
            lov = jnp.full((16,), lo, jnp.int32)
            trash = jnp.full((16,), _FLATW, jnp.int32)

            def inner(j, c2):
                ix = iv[pl.ds(j * 16, 16)]
                vx = vv[pl.ds(j * 16, 16)]
                loc = ix - lov
                m = (ix >= lov) & (loc < trash)
                plsc.store_scatter(wb, [jnp.where(m, loc, trash)], vx)
                return c2

            return lax.fori_loop(0, _CAPP // 16, inner, carry)

        lax.fori_loop(0, _NW, outer, jnp.int32(0))
        pltpu.sync_copy(wb.at[pl.ds(0, _FLATW)], wd_hbm.at[pl.ds(lo, _FLATW)])

    return k(idx_arr, val_arr)


def _build_bias(rows, cols, vals):
    idx_arr, val_arr = _compact_edges(rows, cols, vals)
    return _scatter_bias(idx_arr, val_arr).reshape(_NPAD, DIM)


def _attention(x2d, wd2d, wqkv, bq8, wproj, bp8, gate8):
    n = x2d.shape[0]
    steps = n // DIM
    scale = 1.0 / math.sqrt(HD)

    def body(x_ref, wd_ref, wqkv_ref, bq_ref, wproj_ref, bp_ref, g_ref, o_ref):
        xt = x_ref[...]
        qkv = (
            jnp.dot(xt, wqkv_ref[...], preferred_element_type=jnp.float32)
            + bq_ref[0:1, :]
        )
        gi = lax.broadcasted_iota(jnp.int32, (4, DIM), 0)
        ri = lax.broadcasted_iota(jnp.int32, (4, DIM), 1)
        seg = jnp.where(ri // BS == gi, 1.0 / BS, 0.0)
        qkv_m = jnp.dot(seg, qkv, preferred_element_type=jnp.float32)
        er = lax.broadcasted_iota(jnp.int32, (DIM, 4), 0)
        eg = lax.broadcasted_iota(jnp.int32, (DIM, 4), 1)
        exp4 = jnp.where(er // BS == eg, 1.0, 0.0)

        wd = wd_ref[...]
        rr = lax.broadcasted_iota(jnp.int32, (DIM, DIM), 0)
        cc = lax.broadcasted_iota(jnp.int32, (DIM, DIM), 1)
        eye = rr == cc
        valid = wd == wd
        bias = jnp.where(eye, 1.0, jnp.where(valid, wd, 0.0))
        ok = valid | eye

        outs = []
        for h in range(NH):
            q = qkv[:, h * HD:(h + 1) * HD]
            kx = qkv[:, DIM + h * HD: DIM + (h + 1) * HD]
            vx = qkv[:, 2 * DIM + h * HD: 2 * DIM + (h + 1) * HD]
            km = qkv_m[:, DIM + h * HD: DIM + (h + 1) * HD]
            vm = qkv_m[:, 2 * DIM + h * HD: 2 * DIM + (h + 1) * HD]
            s = (
                lax.dot_general(q, kx, (((1,), (1,)), ((), ())),
                                preferred_element_type=jnp.float32) * scale
                + bias
            )
            s = jnp.where(ok, s, -1e30)
            kme = jnp.dot(exp4, km, preferred_element_type=jnp.float32)
            sm = jnp.sum(q * kme, axis=1, keepdims=True) * scale + 1.0
            mx = jnp.maximum(jnp.max(s, axis=1, keepdims=True), sm)
            px = jnp.exp(s - mx)
            pm = jnp.exp(sm - mx)
            den = jnp.sum(px, axis=1, keepdims=True) + pm
            wg = g_ref[0, h]
            bg = g_ref[1, h]
            cx = px / den + jnp.where(ok, bias * wg + bg, 0.0)
            cm = pm / den + (wg + bg)
            vme = jnp.dot(exp4, vm, preferred_element_type=jnp.float32)
            outs.append(
                jnp.dot(cx, vx, preferred_element_type=jnp.float32) + cm * vme
            )
        attn = jnp.concatenate(outs, axis=1)
        o_ref[...] = (
            jnp.dot(attn, wproj_ref[...], preferred_element_type=jnp.float32)
            + bp_ref[0:1, :]
        )

    return pl.pallas_call(
        body,
        grid=(steps,),
        in_specs=[
            pl.BlockSpec((DIM, DIM), lambda i: (i, 0)),
            pl.BlockSpec((DIM, DIM), lambda i: (i, 0)),
            pl.BlockSpec((DIM, 3 * DIM), lambda i: (0, 0)),
            pl.BlockSpec((8, 3 * DIM), lambda i: (0, 0)),
            pl.BlockSpec((DIM, DIM), lambda i: (0, 0)),
            pl.BlockSpec((8, DIM), lambda i: (0, 0)),
            pl.BlockSpec((8, DIM), lambda i: (0, 0)),
        ],
        out_specs=pl.BlockSpec((DIM, DIM), lambda i: (i, 0)),
        out_shape=jax.ShapeDtypeStruct((n, DIM), jnp.float32),
    )(x2d, wd2d, wqkv, bq8, wproj, bp8, gate8)


def kernel(x, edge_index, edge_values, positions, Wqkv, bqkv, Wproj, bproj,
           Wgate, bgate):
    del positions
    b, n, c = x.shape
    rows = edge_index[0].astype(jnp.int32)
    cols = edge_index[1].astype(jnp.int32)
    vals = edge_values.astype(jnp.float32)
    wd2d = _build_bias(rows, cols, vals)
    x2d = x.reshape(n, c)
    gate8 = (
        jnp.zeros((8, DIM), jnp.float32)
        .at[0, :NH].set(Wgate[0].astype(jnp.float32))
        .at[1, :NH].set(bgate.astype(jnp.float32))
    )
    bq8 = jnp.zeros((8, 3 * DIM), jnp.float32).at[0].set(bqkv)
    bp8 = jnp.zeros((8, DIM), jnp.float32).at[0].set(bproj)
    y = _attention(x2d, wd2d, Wqkv, bq8, Wproj, bp8, gate8)
    return y.reshape(b, n, c)


# speedup vs baseline: 22.5883x; 1.1593x over previous
import functools
import math

import jax
import jax.numpy as jnp
from jax import lax
from jax.experimental import pallas as pl
from jax.experimental.pallas import tpu as pltpu
from jax.experimental.pallas import tpu_sc as plsc

DIM = 128
BS = 32
NH = 4
HD = DIM // NH

_NW = 32
_CAP = 1024
_CAPP = _CAP + 16
_SENT = 2**31 - 1
_BLK_PER_W = 10
_ROWS_W = _BLK_PER_W * BS
_FLATW = _ROWS_W * DIM
_NPAD = _NW * _ROWS_W


def _compact_edges(rows, cols, vals):
    e = rows.shape[0]
    chunk = e // _NW
    nvec = chunk // 16
    mesh = plsc.VectorSubcoreMesh(core_axis_name="c", subcore_axis_name="s")

    @functools.partial(
        pl.kernel,
        out_type=(
            jax.ShapeDtypeStruct((_NW * _CAPP,), jnp.int32),
            jax.ShapeDtypeStruct((_NW * _CAPP,), jnp.float32),
        ),
        mesh=mesh,
        scratch_types=[
            pltpu.VMEM((chunk,), jnp.int32),
            pltpu.VMEM((chunk,), jnp.int32),
            pltpu.VMEM((chunk,), jnp.float32),
            pltpu.VMEM((_CAPP,), jnp.int32),
            pltpu.VMEM((_CAPP,), jnp.float32),
        ],
        compiler_params=pltpu.CompilerParams(needs_layout_passes=False),
    )
    def k(rows_hbm, cols_hbm, vals_hbm, idx_hbm, val_hbm, rv, cv, vv, ib, vb):
        wid = lax.axis_index("s") * 2 + lax.axis_index("c")
        base = wid * chunk
        pltpu.sync_copy(rows_hbm.at[pl.ds(base, chunk)], rv)
        pltpu.sync_copy(cols_hbm.at[pl.ds(base, chunk)], cv)
        pltpu.sync_copy(vals_hbm.at[pl.ds(base, chunk)], vv)

        sent = jnp.full((16,), _SENT, jnp.int32)

        def init(i, carry):
            ib[pl.ds(i * 16, 16)] = sent
            return carry

        lax.fori_loop(0, _CAPP // 16, init, jnp.int32(0))

        def body(i, off):
            r = rv[pl.ds(i * 16, 16)]
            c = cv[pl.ds(i * 16, 16)]
            v = vv[pl.ds(i * 16, 16)]
            br = r >> 5
            intra = br == (c >> 5)
            idx = r * DIM + (br & 3) * BS + (c & 31)
            ii = intra.astype(jnp.int32)
            offv = jnp.full((16,), off, jnp.int32)
            trash = jnp.full((16,), _CAPP - 1, jnp.int32)
            pos = offv + plsc.cumsum(ii) - ii
            tgt = jnp.where(intra, pos, trash)
            plsc.store_scatter(ib, [tgt], jnp.where(intra, idx, sent))
            plsc.store_scatter(vb, [tgt], v)
            cnt = jnp.sum(ii)
            return jnp.minimum(off + cnt, jnp.int32(_CAP))

        lax.fori_loop(0, nvec, body, jnp.int32(0))
        pltpu.sync_copy(ib, idx_hbm.at[pl.ds(wid * _CAPP, _CAPP)])
        pltpu.sync_copy(vb, val_hbm.at[pl.ds(wid * _CAPP, _CAPP)])

    return k(rows, cols, vals)


def _scatter_bias(idx_arr, val_arr):
    mesh = plsc.VectorSubcoreMesh(core_axis_name="c", subcore_axis_name="s")

    @functools.partial(
        pl.kernel,
        out_type=jax.ShapeDtypeStruct((_NPAD * DIM,), jnp.float32),
        mesh=mesh,
        scratch_types=[
            pltpu.VMEM((_NW * _CAPP,), jnp.int32),
            pltpu.VMEM((_NW * _CAPP,), jnp.float32),
            pltpu.VMEM((_FLATW + 16,), jnp.float32),
        ],
        compiler_params=pltpu.CompilerParams(needs_layout_passes=False),
    )
    def k(idx_hbm, val_hbm, wd_hbm, iv, vv, wb):
        wid = lax.axis_index("s") * 2 + lax.axis_index("c")
        lo = wid * _FLATW
        pltpu.sync_copy(idx_hbm, iv)
        pltpu.sync_copy(val_hbm, vv)
        nanv = jnp.full((16,), jnp.nan, jnp.float32)

        def init(i, carry):
            for u in range(4):
                wb[pl.ds((i * 4 + u) * 16, 16)] = nanv
            return carry

        lax.fori_loop(0, _FLATW // 64, init, jnp.int32(0))
        wb[pl.ds(_FLATW, 16)] = nanv

        lov = jnp.full((16,), lo, jnp.int32)
        trash = jnp.full((16,), _FLATW, jnp.int32)

        def inner(j, c2):
            for u in range(4):
                ix = iv[pl.ds((j * 4 + u) * 16, 16)]
                vx = vv[pl.ds((j * 4 + u) * 16, 16)]
                loc = ix - lov
                m = (ix >= lov) & (loc < trash)
                plsc.store_scatter(wb, [jnp.where(m, loc, trash)], vx)
            return c2

        lax.fori_loop(0, _NW * _CAPP // 64, inner, jnp.int32(0))
        pltpu.sync_copy(wb.at[pl.ds(0, _FLATW)], wd_hbm.at[pl.ds(lo, _FLATW)])

    return k(idx_arr, val_arr)


def _build_bias(rows, cols, vals):
    idx_arr, val_arr = _compact_edges(rows, cols, vals)
    return _scatter_bias(idx_arr, val_arr).reshape(_NPAD, DIM)


def _attention(x2d, wd2d, wqkv, bq8, wproj, bp8, gate8):
    n = x2d.shape[0]
    steps = n // DIM
    scale = 1.0 / math.sqrt(HD)

    def body(x_ref, wd_ref, wqkv_ref, bq_ref, wproj_ref, bp_ref, g_ref, o_ref):
        xt = x_ref[...]
        qkv = (
            jnp.dot(xt, wqkv_ref[...], preferred_element_type=jnp.float32)
            + bq_ref[0:1, :]
        )
        gi = lax.broadcasted_iota(jnp.int32, (4, DIM), 0)
        ri = lax.broadcasted_iota(jnp.int32, (4, DIM), 1)
        seg = jnp.where(ri // BS == gi, 1.0 / BS, 0.0)
        qkv_m = jnp.dot(seg, qkv, preferred_element_type=jnp.float32)
        er = lax.broadcasted_iota(jnp.int32, (DIM, 4), 0)
        eg = lax.broadcasted_iota(jnp.int32, (DIM, 4), 1)
        exp4 = jnp.where(er // BS == eg, 1.0, 0.0)

        wd = wd_ref[...]
        rr = lax.broadcasted_iota(jnp.int32, (DIM, DIM), 0)
        cc = lax.broadcasted_iota(jnp.int32, (DIM, DIM), 1)
        eye = rr == cc
        valid = wd == wd
        bias = jnp.where(eye, 1.0, jnp.where(valid, wd, 0.0))
        ok = valid | eye

        outs = []
        for h in range(NH):
            q = qkv[:, h * HD:(h + 1) * HD]
            kx = qkv[:, DIM + h * HD: DIM + (h + 1) * HD]
            vx = qkv[:, 2 * DIM + h * HD: 2 * DIM + (h + 1) * HD]
            km = qkv_m[:, DIM + h * HD: DIM + (h + 1) * HD]
            vm = qkv_m[:, 2 * DIM + h * HD: 2 * DIM + (h + 1) * HD]
            s = (
                lax.dot_general(q, kx, (((1,), (1,)), ((), ())),
                                preferred_element_type=jnp.float32) * scale
                + bias
            )
            s = jnp.where(ok, s, -1e30)
            kme = jnp.dot(exp4, km, preferred_element_type=jnp.float32)
            sm = jnp.sum(q * kme, axis=1, keepdims=True) * scale + 1.0
            mx = jnp.maximum(jnp.max(s, axis=1, keepdims=True), sm)
            px = jnp.exp(s - mx)
            pm = jnp.exp(sm - mx)
            den = jnp.sum(px, axis=1, keepdims=True) + pm
            wg = g_ref[0, h]
            bg = g_ref[1, h]
            cx = px / den + jnp.where(ok, bias * wg + bg, 0.0)
            cm = pm / den + (wg + bg)
            vme = jnp.dot(exp4, vm, preferred_element_type=jnp.float32)
            outs.append(
                jnp.dot(cx, vx, preferred_element_type=jnp.float32) + cm * vme
            )
        attn = jnp.concatenate(outs, axis=1)
        o_ref[...] = (
            jnp.dot(attn, wproj_ref[...], preferred_element_type=jnp.float32)
            + bp_ref[0:1, :]
        )

    return pl.pallas_call(
        body,
        grid=(steps,),
        in_specs=[
            pl.BlockSpec((DIM, DIM), lambda i: (i, 0)),
            pl.BlockSpec((DIM, DIM), lambda i: (i, 0)),
            pl.BlockSpec((DIM, 3 * DIM), lambda i: (0, 0)),
            pl.BlockSpec((8, 3 * DIM), lambda i: (0, 0)),
            pl.BlockSpec((DIM, DIM), lambda i: (0, 0)),
            pl.BlockSpec((8, DIM), lambda i: (0, 0)),
            pl.BlockSpec((8, DIM), lambda i: (0, 0)),
        ],
        out_specs=pl.BlockSpec((DIM, DIM), lambda i: (i, 0)),
        out_shape=jax.ShapeDtypeStruct((n, DIM), jnp.float32),
    )(x2d, wd2d, wqkv, bq8, wproj, bp8, gate8)


def kernel(x, edge_index, edge_values, positions, Wqkv, bqkv, Wproj, bproj,
           Wgate, bgate):
    del positions
    b, n, c = x.shape
    rows = edge_index[0].astype(jnp.int32)
    cols = edge_index[1].astype(jnp.int32)
    vals = edge_values.astype(jnp.float32)
    wd2d = _build_bias(rows, cols, vals)
    x2d = x.reshape(n, c)
    gate8 = (
        jnp.zeros((8, DIM), jnp.float32)
        .at[0, :NH].set(Wgate[0].astype(jnp.float32))
        .at[1, :NH].set(bgate.astype(jnp.float32))
    )
    bq8 = jnp.zeros((8, 3 * DIM), jnp.float32).at[0].set(bqkv)
    bp8 = jnp.zeros((8, DIM), jnp.float32).at[0].set(bproj)
    y = _attention(x2d, wd2d, Wqkv, bq8, Wproj, bp8, gate8)
    return y.reshape(b, n, c)
